# baseline (speedup 1.0000x reference)
import functools

import jax
import jax.numpy as jnp
from jax import lax
from jax.experimental import pallas as pl
from jax.experimental.pallas import tpu as pltpu
from jax.experimental.pallas import tpu_sc as plsc

N = 10000
NP = 10240
E = 320000
D_IN = 128
HID = 128
NCLS = 40
NC_PAD = 128

NW = 32
B = 128
TOTB = 2560
CH = 8
EPAD = TOTB * B - E
ROWS = NP // 16
K0 = 80
K1 = 80

_mesh = functools.partial(
    plsc.VectorSubcoreMesh, core_axis_name="c", subcore_axis_name="s")


def _make_agg(D):
  @functools.partial(
      pl.kernel,
      out_type=jax.ShapeDtypeStruct((2, NP, D), jnp.float32),
      mesh=_mesh(),
      scratch_types=[
          pltpu.VMEM((2, CH, B), jnp.int32),
          pltpu.VMEM((2, CH, B), jnp.int32),
          pltpu.VMEM((2, B, D), jnp.float32),
          pltpu.VMEM_SHARED((NP, D), jnp.float32),
          pltpu.SemaphoreType.DMA((2,)),
      ],
  )
  def agg(g_hbm, src_hbm, dst_hbm, zero_hbm, out_hbm,
          src_v, dst_v, rows_v, acc, sems):
    c = lax.axis_index("c")
    s = lax.axis_index("s")
    w = c * 16 + s
    r0 = s * ROWS

    @pl.when(c == 0)
    def _():
      pltpu.sync_copy(g_hbm.at[pl.ds(r0, ROWS)], acc.at[pl.ds(r0, ROWS)])

    @pl.when(c != 0)
    def _():
      pltpu.sync_copy(zero_hbm.at[pl.ds(r0, ROWS)], acc.at[pl.ds(r0, ROWS)])

    nb = TOTB // NW
    nch = nb // CH

    pltpu.sync_copy(src_hbm.at[w, pl.ds(0, CH)], src_v.at[0])
    pltpu.sync_copy(dst_hbm.at[w, pl.ds(0, CH)], dst_v.at[0])
    plsc.subcore_barrier()

    pltpu.async_copy(g_hbm.at[src_v.at[0, 0]], rows_v.at[0], sems.at[0])

    def outer(ch, carry):
      @pl.when(ch + 1 < nch)
      def _():
        pltpu.sync_copy(src_hbm.at[w, pl.ds((ch + 1) * CH, CH)],
                        src_v.at[lax.rem(ch + 1, 2)])
        pltpu.sync_copy(dst_hbm.at[w, pl.ds((ch + 1) * CH, CH)],
                        dst_v.at[lax.rem(ch + 1, 2)])

      def inner(j, carry2):
        i = ch * CH + j
        n = i + 1

        @pl.when(n < nb)
        def _():
          pltpu.async_copy(
              g_hbm.at[src_v.at[lax.rem(n // CH, 2), lax.rem(n, CH)]],
              rows_v.at[lax.rem(n, 2)], sems.at[lax.rem(n, 2)])

        p = lax.rem(i, 2)
        pltpu.make_async_copy(g_hbm.at[src_v.at[0, 0]], rows_v.at[p],
                              sems.at[p]).wait()
        pltpu.sync_copy(rows_v.at[p], acc.at[dst_v.at[lax.rem(ch, 2), j]],
                        add=True)
        return carry2

      lax.fori_loop(0, CH, inner, 0)
      return carry

    lax.fori_loop(0, nch, outer, 0)
    plsc.subcore_barrier()
    pltpu.sync_copy(acc.at[pl.ds(r0, ROWS)], out_hbm.at[c, pl.ds(r0, ROWS)])

  return agg


_agg_hid = _make_agg(HID)
_agg_cls = _make_agg(NC_PAD)


@functools.partial(
    pl.kernel,
    out_type=jax.ShapeDtypeStruct((2, NP, 16), jnp.float32),
    mesh=_mesh(),
    scratch_types=[
        pltpu.VMEM((TOTB // NW, B), jnp.int32),
        pltpu.VMEM((B, 16), jnp.float32),
        pltpu.VMEM_SHARED((NP, 16), jnp.float32),
    ],
    compiler_params=pltpu.CompilerParams(use_tc_tiling_on_sc=False),
)
def _deg(dst_hbm, ones_hbm, zero_hbm, out_hbm, dst_v, ones_v, acc):
  c = lax.axis_index("c")
  s = lax.axis_index("s")
  w = c * 16 + s
  r0 = s * ROWS

  pltpu.sync_copy(zero_hbm.at[pl.ds(r0, ROWS)], acc.at[pl.ds(r0, ROWS)])
  pltpu.sync_copy(ones_hbm, ones_v)
  nb = TOTB // NW
  pltpu.sync_copy(dst_hbm.at[w], dst_v)
  plsc.subcore_barrier()

  def body(i, carry):
    pltpu.sync_copy(ones_v, acc.at[dst_v.at[i]], add=True)
    return carry

  lax.fori_loop(0, nb, body, 0)
  plsc.subcore_barrier()
  pltpu.sync_copy(acc.at[pl.ds(r0, ROWS)], out_hbm.at[c, pl.ds(r0, ROWS)])


def _dinv_from_degp(degp_blk):
  deg = jnp.sum(degp_blk, axis=(0, 2)) * (1.0 / 16.0) + 1.0
  return lax.rsqrt(deg)


def _tc1a_body(x_ref, w_ref, h_ref):
  h_ref[...] = jnp.dot(x_ref[...], w_ref[...],
                       preferred_element_type=jnp.float32)


def _tc1b_body(h_ref, degp_ref, g_ref):
  dinv = _dinv_from_degp(degp_ref[...])
  g_ref[...] = h_ref[...] * dinv[:, None]


def _tc2_body(p_ref, degp_ref, b1_ref, w2_ref, g2_ref):
  dinv = _dinv_from_degp(degp_ref[...])
  y = (p_ref[0] + p_ref[1]) * dinv[:, None] + b1_ref[...]
  h = jnp.maximum(y, 0.0)
  g2_ref[...] = jnp.dot(h, w2_ref[...],
                        preferred_element_type=jnp.float32) * dinv[:, None]


def _tc3_body(p_ref, degp_ref, b2_ref, o_ref):
  dinv = _dinv_from_degp(degp_ref[...])
  y = (p_ref[0] + p_ref[1]) * dinv[:, None] + b2_ref[...]
  lanes = lax.broadcasted_iota(jnp.int32, y.shape, 1)
  ym = jnp.where(lanes < NCLS, y, -1e30)
  m = jnp.max(ym, axis=1, keepdims=True)
  lse = jnp.log(jnp.sum(jnp.exp(ym - m), axis=1, keepdims=True))
  o_ref[...] = y - m - lse


_BR = 512
_GRID = (NP // _BR,)


def _tc1a(x_pad, W1):
  return pl.pallas_call(
      _tc1a_body,
      grid=_GRID,
      in_specs=[
          pl.BlockSpec((_BR, D_IN), lambda i: (i, 0)),
          pl.BlockSpec((D_IN, HID), lambda i: (0, 0)),
      ],
      out_specs=pl.BlockSpec((_BR, HID), lambda i: (i, 0)),
      out_shape=jax.ShapeDtypeStruct((NP, HID), jnp.float32),
  )(x_pad, W1)


def _tc1b(h1, degp):
  return pl.pallas_call(
      _tc1b_body,
      grid=_GRID,
      in_specs=[
          pl.BlockSpec((_BR, HID), lambda i: (i, 0)),
          pl.BlockSpec((2, _BR, 16), lambda i: (0, i, 0)),
      ],
      out_specs=pl.BlockSpec((_BR, HID), lambda i: (i, 0)),
      out_shape=jax.ShapeDtypeStruct((NP, HID), jnp.float32),
  )(h1, degp)


def _tc2(p1, degp, b1r, W2p):
  return pl.pallas_call(
      _tc2_body,
      grid=_GRID,
      in_specs=[
          pl.BlockSpec((2, _BR, HID), lambda i: (0, i, 0)),
          pl.BlockSpec((2, _BR, 16), lambda i: (0, i, 0)),
          pl.BlockSpec((1, HID), lambda i: (0, 0)),
          pl.BlockSpec((HID, NC_PAD), lambda i: (0, 0)),
      ],
      out_specs=pl.BlockSpec((_BR, NC_PAD), lambda i: (i, 0)),
      out_shape=jax.ShapeDtypeStruct((NP, NC_PAD), jnp.float32),
  )(p1, degp, b1r, W2p)


def _tc3(p2, degp, b2r):
  return pl.pallas_call(
      _tc3_body,
      grid=_GRID,
      in_specs=[
          pl.BlockSpec((2, _BR, NC_PAD), lambda i: (0, i, 0)),
          pl.BlockSpec((2, _BR, 16), lambda i: (0, i, 0)),
          pl.BlockSpec((1, NC_PAD), lambda i: (0, 0)),
      ],
      out_specs=pl.BlockSpec((_BR, NC_PAD), lambda i: (i, 0)),
      out_shape=jax.ShapeDtypeStruct((NP, NC_PAD), jnp.float32),
  )(p2, degp, b2r)


@jax.jit
def kernel(x, edge_index, W1, b1, W2, b2):
  ei = edge_index.astype(jnp.int32)
  pad_dst = N + jnp.arange(EPAD, dtype=jnp.int32) % (NP - N)
  src = jnp.concatenate([ei[0], jnp.zeros((EPAD,), jnp.int32)])
  dst = jnp.concatenate([ei[1], pad_dst])
  src3 = src.reshape(NW, TOTB // NW, B)
  dst3 = dst.reshape(NW, TOTB // NW, B)

  x_pad = jnp.pad(x, ((0, NP - N), (0, 0)))
  W2p = jnp.pad(W2, ((0, 0), (0, NC_PAD - NCLS)))
  b1r = b1[None, :]
  b2r = jnp.pad(b2, (0, NC_PAD - NCLS))[None, :]

  ones16 = jnp.ones((B, 16), jnp.float32)
  z16 = jnp.zeros((NP, 16), jnp.float32)
  zhid = jnp.zeros((NP, HID), jnp.float32)
  zcls = jnp.zeros((NP, NC_PAD), jnp.float32)

  degp = _deg(dst3, ones16, z16)
  h1 = _tc1a(x_pad, W1)
  g1 = _tc1b(h1, degp)
  p1 = _agg_hid(g1, src3, dst3, zhid)
  g2 = _tc2(p1, degp, b1r, W2p)
  p2 = _agg_cls(g2, src3, dst3, zcls)
  outp = _tc3(p2, degp, b2r)
  return outp[:N, :NCLS]


# speedup vs baseline: 1.0171x; 1.0171x over previous
import functools

import jax
import jax.numpy as jnp
from jax import lax
from jax.experimental import pallas as pl
from jax.experimental.pallas import tpu as pltpu
from jax.experimental.pallas import tpu_sc as plsc

N = 10000
NP = 10240
E = 320000
D_IN = 128
HID = 128
NCLS = 40
NC_PAD = 128

NW = 32
B = 128
TOTB = 2560
CH = 8
EPAD = TOTB * B - E
ROWS = NP // 16
K0 = 80
K1 = 80

_mesh = functools.partial(
    plsc.VectorSubcoreMesh, core_axis_name="c", subcore_axis_name="s")


def _make_agg(D):
  @functools.partial(
      pl.kernel,
      out_type=jax.ShapeDtypeStruct((2, NP, D), jnp.float32),
      mesh=_mesh(),
      scratch_types=[
          pltpu.VMEM((2, CH, B), jnp.int32),
          pltpu.VMEM((2, CH, B), jnp.int32),
          pltpu.VMEM((2, B, D), jnp.float32),
          pltpu.VMEM_SHARED((NP, D), jnp.float32),
          pltpu.SemaphoreType.DMA((2,)),
      ],
  )
  def agg(g_hbm, src_hbm, dst_hbm, zero_hbm, out_hbm,
          src_v, dst_v, rows_v, acc, sems):
    c = lax.axis_index("c")
    s = lax.axis_index("s")
    w = c * 16 + s
    r0 = s * ROWS

    @pl.when(c == 0)
    def _():
      pltpu.sync_copy(g_hbm.at[pl.ds(r0, ROWS)], acc.at[pl.ds(r0, ROWS)])

    @pl.when(c != 0)
    def _():
      pltpu.sync_copy(zero_hbm.at[pl.ds(r0, ROWS)], acc.at[pl.ds(r0, ROWS)])

    nb = TOTB // NW
    nch = nb // CH

    pltpu.sync_copy(src_hbm.at[w, pl.ds(0, CH)], src_v.at[0])
    pltpu.sync_copy(dst_hbm.at[w, pl.ds(0, CH)], dst_v.at[0])
    plsc.subcore_barrier()

    pltpu.async_copy(g_hbm.at[src_v.at[0, 0]], rows_v.at[0], sems.at[0])

    def outer(ch, carry):
      @pl.when(ch + 1 < nch)
      def _():
        pltpu.sync_copy(src_hbm.at[w, pl.ds((ch + 1) * CH, CH)],
                        src_v.at[lax.rem(ch + 1, 2)])
        pltpu.sync_copy(dst_hbm.at[w, pl.ds((ch + 1) * CH, CH)],
                        dst_v.at[lax.rem(ch + 1, 2)])

      def inner(j, carry2):
        i = ch * CH + j
        n = i + 1

        @pl.when(n < nb)
        def _():
          pltpu.async_copy(
              g_hbm.at[src_v.at[lax.rem(n // CH, 2), lax.rem(n, CH)]],
              rows_v.at[lax.rem(n, 2)], sems.at[lax.rem(n, 2)])

        p = lax.rem(i, 2)
        pltpu.make_async_copy(g_hbm.at[src_v.at[0, 0]], rows_v.at[p],
                              sems.at[p]).wait()
        pltpu.sync_copy(rows_v.at[p], acc.at[dst_v.at[lax.rem(ch, 2), j]],
                        add=True)
        return carry2

      lax.fori_loop(0, CH, inner, 0)
      return carry

    lax.fori_loop(0, nch, outer, 0)
    plsc.subcore_barrier()
    pltpu.sync_copy(acc.at[pl.ds(r0, ROWS)], out_hbm.at[c, pl.ds(r0, ROWS)])

  return agg


_agg_hid = _make_agg(HID)
_agg_cls = _make_agg(NC_PAD)


@functools.partial(
    pl.kernel,
    out_type=jax.ShapeDtypeStruct((2, NP, 16), jnp.float32),
    mesh=_mesh(),
    scratch_types=[
        pltpu.VMEM((TOTB // NW, B), jnp.int32),
        pltpu.VMEM((B, 16), jnp.float32),
        pltpu.VMEM_SHARED((NP, 16), jnp.float32),
    ],
    compiler_params=pltpu.CompilerParams(use_tc_tiling_on_sc=False),
)
def _deg(dst_hbm, ones_hbm, zero_hbm, out_hbm, dst_v, ones_v, acc):
  c = lax.axis_index("c")
  s = lax.axis_index("s")
  w = c * 16 + s
  r0 = s * ROWS

  pltpu.sync_copy(zero_hbm.at[pl.ds(r0, ROWS)], acc.at[pl.ds(r0, ROWS)])
  pltpu.sync_copy(ones_hbm, ones_v)
  nb = TOTB // NW
  pltpu.sync_copy(dst_hbm.at[w], dst_v)
  plsc.subcore_barrier()

  def body(i, carry):
    pltpu.sync_copy(ones_v, acc.at[dst_v.at[i]], add=True)
    return carry

  lax.fori_loop(0, nb, body, 0)
  plsc.subcore_barrier()
  pltpu.sync_copy(acc.at[pl.ds(r0, ROWS)], out_hbm.at[c, pl.ds(r0, ROWS)])


def _dinv_from_degp(degp_blk):
  deg = jnp.sum(degp_blk, axis=(0, 2)) * (1.0 / 16.0) + 1.0
  return lax.rsqrt(deg)


def _tc1_body(x_ref, w_ref, degp_ref, g_ref):
  dinv = _dinv_from_degp(degp_ref[...])
  h = jnp.dot(x_ref[...], w_ref[...], preferred_element_type=jnp.float32)
  g_ref[...] = h * dinv[:, None]


def _tc2_body(p_ref, degp_ref, b1_ref, w2_ref, g2_ref):
  dinv = _dinv_from_degp(degp_ref[...])
  y = (p_ref[0] + p_ref[1]) * dinv[:, None] + b1_ref[...]
  h = jnp.maximum(y, 0.0)
  g2_ref[...] = jnp.dot(h, w2_ref[...],
                        preferred_element_type=jnp.float32) * dinv[:, None]


def _tc3_body(p_ref, degp_ref, b2_ref, o_ref):
  dinv = _dinv_from_degp(degp_ref[...])
  y = (p_ref[0] + p_ref[1]) * dinv[:, None] + b2_ref[...]
  lanes = lax.broadcasted_iota(jnp.int32, y.shape, 1)
  ym = jnp.where(lanes < NCLS, y, -1e30)
  m = jnp.max(ym, axis=1, keepdims=True)
  lse = jnp.log(jnp.sum(jnp.exp(ym - m), axis=1, keepdims=True))
  o_ref[...] = y - m - lse


_BR = 512
_GRID = (NP // _BR,)


def _tc1(x_pad, W1, degp):
  return pl.pallas_call(
      _tc1_body,
      grid=_GRID,
      in_specs=[
          pl.BlockSpec((_BR, D_IN), lambda i: (i, 0)),
          pl.BlockSpec((D_IN, HID), lambda i: (0, 0)),
          pl.BlockSpec((2, _BR, 16), lambda i: (0, i, 0)),
      ],
      out_specs=pl.BlockSpec((_BR, HID), lambda i: (i, 0)),
      out_shape=jax.ShapeDtypeStruct((NP, HID), jnp.float32),
  )(x_pad, W1, degp)


def _tc2(p1, degp, b1r, W2p):
  return pl.pallas_call(
      _tc2_body,
      grid=_GRID,
      in_specs=[
          pl.BlockSpec((2, _BR, HID), lambda i: (0, i, 0)),
          pl.BlockSpec((2, _BR, 16), lambda i: (0, i, 0)),
          pl.BlockSpec((1, HID), lambda i: (0, 0)),
          pl.BlockSpec((HID, NC_PAD), lambda i: (0, 0)),
      ],
      out_specs=pl.BlockSpec((_BR, NC_PAD), lambda i: (i, 0)),
      out_shape=jax.ShapeDtypeStruct((NP, NC_PAD), jnp.float32),
  )(p1, degp, b1r, W2p)


def _tc3(p2, degp, b2r):
  return pl.pallas_call(
      _tc3_body,
      grid=_GRID,
      in_specs=[
          pl.BlockSpec((2, _BR, NC_PAD), lambda i: (0, i, 0)),
          pl.BlockSpec((2, _BR, 16), lambda i: (0, i, 0)),
          pl.BlockSpec((1, NC_PAD), lambda i: (0, 0)),
      ],
      out_specs=pl.BlockSpec((_BR, NC_PAD), lambda i: (i, 0)),
      out_shape=jax.ShapeDtypeStruct((NP, NC_PAD), jnp.float32),
  )(p2, degp, b2r)


@jax.jit
def kernel(x, edge_index, W1, b1, W2, b2):
  ei = edge_index.astype(jnp.int32)
  pad_dst = N + jnp.arange(EPAD, dtype=jnp.int32) % (NP - N)
  src = jnp.concatenate([ei[0], jnp.zeros((EPAD,), jnp.int32)])
  dst = jnp.concatenate([ei[1], pad_dst])
  src3 = src.reshape(NW, TOTB // NW, B)
  dst3 = dst.reshape(NW, TOTB // NW, B)

  x_pad = jnp.pad(x, ((0, NP - N), (0, 0)))
  W2p = jnp.pad(W2, ((0, 0), (0, NC_PAD - NCLS)))
  b1r = b1[None, :]
  b2r = jnp.pad(b2, (0, NC_PAD - NCLS))[None, :]

  ones16 = jnp.ones((B, 16), jnp.float32)
  z16 = jnp.zeros((NP, 16), jnp.float32)
  zhid = jnp.zeros((NP, HID), jnp.float32)
  zcls = jnp.zeros((NP, NC_PAD), jnp.float32)

  degp = _deg(dst3, ones16, z16)
  g1 = _tc1(x_pad, W1, degp)
  p1 = _agg_hid(g1, src3, dst3, zhid)
  g2 = _tc2(p1, degp, b1r, W2p)
  p2 = _agg_cls(g2, src3, dst3, zcls)
  outp = _tc3(p2, degp, b2r)
  return outp[:N, :NCLS]
